# baseline (speedup 1.0000x reference)
import functools

import jax
import jax.numpy as jnp
from jax import lax
from jax.experimental import pallas as pl
from jax.experimental.pallas import tpu as pltpu
from jax.experimental.pallas import tpu_sc as plsc

BATCH = 16384
D = 64
NC = 2
NS = 16
L = 16
NW = NC * NS
BPW = BATCH // NW
GCHUNK = 128


def _sc_body(uidx_hbm, pidx_hbm, nidx_hbm, utab_hbm, itab_hbm,
             pos_out, neg_out, sq_out,
             uidx_v, pidx_v, nidx_v, urows, prows, nrows,
             posb, negb, sqb, sem_u, sem_p, sem_n):
    wid = lax.axis_index("s") * NC + lax.axis_index("c")
    base = wid * BPW

    pltpu.sync_copy(uidx_hbm.at[pl.ds(base, BPW)], uidx_v)
    pltpu.sync_copy(pidx_hbm.at[pl.ds(base, BPW)], pidx_v)
    pltpu.sync_copy(nidx_hbm.at[pl.ds(base, BPW)], nidx_v)

    urows2, prows2, nrows2 = urows, prows, nrows
    copies = []
    for j in range(BPW // GCHUNK):
        sl = pl.ds(j * GCHUNK, GCHUNK)
        copies.append(pltpu.async_copy(utab_hbm.at[uidx_v.at[sl]], urows2.at[sl], sem_u))
        copies.append(pltpu.async_copy(itab_hbm.at[pidx_v.at[sl]], prows2.at[sl], sem_p))
        copies.append(pltpu.async_copy(itab_hbm.at[nidx_v.at[sl]], nrows2.at[sl], sem_n))
    for c in copies:
        c.wait()

    lanes = lax.iota(jnp.int32, L)
    zero = jnp.zeros((L,), jnp.float32)

    def group_body(g, _):
        row16 = g * L + lanes

        def d_body(dd, carry):
            idx, ap, an, asq = carry
            u = plsc.load_gather(urows, [row16, idx])
            p = plsc.load_gather(prows, [row16, idx])
            q = plsc.load_gather(nrows, [row16, idx])
            ap = ap + u * p
            an = an + u * q
            asq = asq + (u * u + (p * p + q * q))
            return (idx + 1, ap, an, asq)

        col0 = jnp.zeros((L,), jnp.int32)
        _, ap, an, asq = lax.fori_loop(0, D, d_body, (col0, zero, zero, zero))
        sl = pl.ds(g * L, L)
        posb[sl] = ap
        negb[sl] = an
        sqb[sl] = asq * 0.5
        return 0

    lax.fori_loop(0, BPW // L, group_body, 0)

    pltpu.sync_copy(posb, pos_out.at[pl.ds(base, BPW)])
    pltpu.sync_copy(negb, neg_out.at[pl.ds(base, BPW)])
    pltpu.sync_copy(sqb, sq_out.at[pl.ds(base, BPW)])


_sc_dots = functools.partial(
    pl.kernel,
    out_type=[
        jax.ShapeDtypeStruct((BATCH,), jnp.float32),
        jax.ShapeDtypeStruct((BATCH,), jnp.float32),
        jax.ShapeDtypeStruct((BATCH,), jnp.float32),
    ],
    mesh=plsc.VectorSubcoreMesh(
        core_axis_name="c", subcore_axis_name="s", num_cores=NC, num_subcores=NS
    ),
    compiler_params=pltpu.CompilerParams(
        needs_layout_passes=False, use_tc_tiling_on_sc=False
    ),
    scratch_types=[
        pltpu.VMEM((BPW,), jnp.int32),
        pltpu.VMEM((BPW,), jnp.int32),
        pltpu.VMEM((BPW,), jnp.int32),
        pltpu.VMEM((BPW, D), jnp.float32),
        pltpu.VMEM((BPW, D), jnp.float32),
        pltpu.VMEM((BPW, D), jnp.float32),
        pltpu.VMEM((BPW,), jnp.float32),
        pltpu.VMEM((BPW,), jnp.float32),
        pltpu.VMEM((BPW,), jnp.float32),
        pltpu.SemaphoreType.DMA,
        pltpu.SemaphoreType.DMA,
        pltpu.SemaphoreType.DMA,
    ],
)(_sc_body)


def _tc_loss_body(pos_ref, neg_ref, sq_ref, out_ref):
    pos = pos_ref[...]
    neg = neg_ref[...]
    sp = 1.0 / (1.0 + jnp.exp(-pos))
    sn = 1.0 / (1.0 + jnp.exp(-neg))
    z = sp - sn
    cf = jnp.mean(jnp.log(1.0 + jnp.exp(-z)))
    reg = jnp.mean(sq_ref[...])
    out_ref[0, 0] = cf + 1e-4 * reg


def kernel(user_indices, pos_item_indices, neg_item_indices, user_table, item_table):
    uidx = user_indices.astype(jnp.int32)
    pidx = pos_item_indices.astype(jnp.int32)
    nidx = neg_item_indices.astype(jnp.int32)

    pos_d, neg_d, sq_d = _sc_dots(uidx, pidx, nidx, user_table, item_table)

    loss = pl.pallas_call(
        _tc_loss_body,
        out_shape=jax.ShapeDtypeStruct((1, 1), jnp.float32),
        out_specs=pl.BlockSpec(memory_space=pltpu.SMEM),
    )(
        pos_d.reshape(128, 128),
        neg_d.reshape(128, 128),
        sq_d.reshape(128, 128),
    )
    return loss[0, 0]
